# Initial kernel scaffold; baseline (speedup 1.0000x reference)
import jax
import jax.numpy as jnp
from jax.experimental import pallas as pl


def kernel(conv1_w, conv1_b, conv2_w, conv2_b, conv3_w, conv3_b, conv4_w, conv4_b, fc1_w, fc1_b, fc2_w, fc2_b, x_nchw):
    raise NotImplementedError("write your pallas kernel here")


# speedup vs baseline: 2.5813x; 2.5813x over previous
import functools

import jax
import jax.numpy as jnp
from jax.experimental import pallas as pl
from jax.experimental.pallas import tpu as pltpu

_PITCH = 32
_HROWS = 34
_R = _PITCH * _HROWS


def _conv_taps(x2d, w_ref, b_ref, rows):
    acc = None
    for dh in range(3):
        grp = None
        for dw in range(3):
            s = dh * _PITCH + dw
            d = jnp.dot(x2d[s:s + rows], w_ref[3 * dh + dw],
                        preferred_element_type=jnp.float32)
            grp = d if grp is None else grp + d
        acc = grp if acc is None else acc + grp
    return jnp.maximum(acc + b_ref[...], 0.0)


def _repad(y, off, h0, h1, w0, w1, batch):
    rows, c = y.shape
    total = batch * _R
    z = jnp.concatenate(
        [jnp.zeros((off, c), y.dtype), y,
         jnp.zeros((total - off - rows, c), y.dtype)], axis=0)
    z4 = z.reshape(batch, _HROWS, _PITCH, c)
    hh = jax.lax.broadcasted_iota(jnp.int32, (batch, _HROWS, _PITCH, 1), 1)
    ww = jax.lax.broadcasted_iota(jnp.int32, (batch, _HROWS, _PITCH, 1), 2)
    m = (hh >= h0) & (hh < h1) & (ww >= w0) & (ww < w1)
    z4 = jnp.where(m, z4, jnp.zeros_like(z4))
    return z4.reshape(total, c)


def _convnet_kernel(x_ref, w1_ref, b1_ref, w2_ref, b2_ref, w3_ref, b3_ref,
                    w4_ref, b4_ref, o_ref, *, batch):
    total = batch * _R
    mc = total - 72
    mp2 = total - 112
    mp4 = total - 144

    x = x_ref[...].reshape(total, x_ref.shape[2])
    y1 = _conv_taps(x, w1_ref, b1_ref, mc).astype(jnp.bfloat16)

    x2 = _repad(y1, 2 * _PITCH + 2, 2, 30, 2, 30, batch)
    y2 = _conv_taps(x2, w2_ref, b2_ref, mc)
    y2p = jnp.maximum(jnp.maximum(y2[0:mp2], y2[1:mp2 + 1]),
                      jnp.maximum(y2[_PITCH:mp2 + _PITCH],
                                  y2[_PITCH + 1:mp2 + _PITCH + 1]))
    y2p = y2p.astype(jnp.bfloat16)

    x3 = _repad(y2p, _PITCH + 1, 1, 30, 1, 30, batch)
    y3 = _conv_taps(x3, w3_ref, b3_ref, mc).astype(jnp.bfloat16)

    x4 = _repad(y3, _PITCH + 1, 1, 30, 1, 30, batch)
    y4 = _conv_taps(x4, w4_ref, b4_ref, mc)
    y4p = jnp.maximum(jnp.maximum(y4[0:mp4], y4[2:mp4 + 2]),
                      jnp.maximum(y4[2 * _PITCH:mp4 + 2 * _PITCH],
                                  y4[2 * _PITCH + 2:mp4 + 2 * _PITCH + 2]))

    c4 = y4p.shape[1]
    y4z = jnp.concatenate(
        [y4p, jnp.zeros((total - mp4, c4), y4p.dtype)], axis=0)
    y6 = y4z.reshape(batch, 17, 2, 16, 2, c4)
    sub = y6[:, :14, 0, :14, 0, :]
    o_ref[...] = sub.reshape(batch, 196, c4).astype(o_ref.dtype)


def _fc_kernel(x_ref, w1_ref, b1_ref, w2_ref, b2_ref, o_ref):
    h = jnp.dot(x_ref[...], w1_ref[...], preferred_element_type=jnp.float32)
    h = jnp.maximum(h + b1_ref[...], 0.0).astype(jnp.bfloat16)
    o = jnp.dot(h, w2_ref[...], preferred_element_type=jnp.float32)
    o_ref[...] = o + b2_ref[...]


def kernel(conv1_w, conv1_b, conv2_w, conv2_b, conv3_w, conv3_b, conv4_w,
           conv4_b, fc1_w, fc1_b, fc2_w, fc2_b, x_nchw):
    n = x_nchw.shape[0]
    batch = 8
    assert n % batch == 0

    xi = x_nchw.reshape(n, 28, 28).astype(jnp.float32)
    xi = jnp.pad(xi, ((0, 0), (1, 1), (1, 3)))
    xi = xi.reshape(n, 30 * 32)
    xi = jnp.pad(xi, ((0, 0), (0, _R - 30 * 32)))
    xi = jnp.pad(xi[..., None], ((0, 0), (0, 0), (0, 7)))
    xi = xi.astype(jnp.bfloat16)

    w1 = jnp.pad(conv1_w, ((0, 0), (0, 0), (0, 7), (0, 0)))
    w1 = w1.reshape(9, 8, 32).astype(jnp.bfloat16)
    w2 = conv2_w.reshape(9, 32, 32).astype(jnp.bfloat16)
    w3 = conv3_w.reshape(9, 32, 64).astype(jnp.bfloat16)
    w4 = conv4_w.reshape(9, 64, 32).astype(jnp.bfloat16)
    b1 = conv1_b.reshape(1, 32)
    b2 = conv2_b.reshape(1, 32)
    b3 = conv3_b.reshape(1, 64)
    b4 = conv4_b.reshape(1, 32)

    feats = pl.pallas_call(
        functools.partial(_convnet_kernel, batch=batch),
        out_shape=jax.ShapeDtypeStruct((n, 196, 32), jnp.bfloat16),
        grid_spec=pltpu.PrefetchScalarGridSpec(
            num_scalar_prefetch=0,
            grid=(n // batch,),
            in_specs=[
                pl.BlockSpec((batch, _R, 8), lambda i: (i, 0, 0)),
                pl.BlockSpec((9, 8, 32), lambda i: (0, 0, 0)),
                pl.BlockSpec((1, 32), lambda i: (0, 0)),
                pl.BlockSpec((9, 32, 32), lambda i: (0, 0, 0)),
                pl.BlockSpec((1, 32), lambda i: (0, 0)),
                pl.BlockSpec((9, 32, 64), lambda i: (0, 0, 0)),
                pl.BlockSpec((1, 64), lambda i: (0, 0)),
                pl.BlockSpec((9, 64, 32), lambda i: (0, 0, 0)),
                pl.BlockSpec((1, 32), lambda i: (0, 0)),
            ],
            out_specs=pl.BlockSpec((batch, 196, 32), lambda i: (i, 0, 0)),
        ),
        compiler_params=pltpu.CompilerParams(
            dimension_semantics=("parallel",)),
    )(xi, w1, b1, w2, b2, w3, b3, w4, b4)

    xf = feats.reshape(n, 6272)
    w1p = fc1_w.reshape(32, 196, 512).transpose(1, 0, 2).reshape(6272, 512)
    w1p = w1p.astype(jnp.bfloat16)
    w2p = jnp.pad(fc2_w, ((0, 0), (0, 118))).astype(jnp.bfloat16)
    b2p = jnp.pad(fc2_b, (0, 118)).reshape(1, 128)
    b1f = fc1_b.reshape(1, 512)

    tm = min(256, n)
    assert n % tm == 0
    out = pl.pallas_call(
        _fc_kernel,
        out_shape=jax.ShapeDtypeStruct((n, 128), jnp.float32),
        grid_spec=pltpu.PrefetchScalarGridSpec(
            num_scalar_prefetch=0,
            grid=(n // tm,),
            in_specs=[
                pl.BlockSpec((tm, 6272), lambda i: (i, 0)),
                pl.BlockSpec((6272, 512), lambda i: (0, 0)),
                pl.BlockSpec((1, 512), lambda i: (0, 0)),
                pl.BlockSpec((512, 128), lambda i: (0, 0)),
                pl.BlockSpec((1, 128), lambda i: (0, 0)),
            ],
            out_specs=pl.BlockSpec((tm, 128), lambda i: (i, 0)),
        ),
        compiler_params=pltpu.CompilerParams(
            dimension_semantics=("parallel",)),
    )(xf, w1p, b1f, w2p, b2p)

    return out[:, :10]


# speedup vs baseline: 6.8019x; 2.6351x over previous
import functools

import jax
import jax.numpy as jnp
from jax.experimental import pallas as pl
from jax.experimental.pallas import tpu as pltpu

_PITCH = 32
_HROWS = 34
_R = _PITCH * _HROWS
_L = 4


def _conv_taps(x2d, w_ref, b_ref, rows):
    acc = None
    for dh in range(3):
        grp = None
        for dw in range(3):
            s = dh * _PITCH + dw
            d = jnp.dot(x2d[s:s + rows], w_ref[3 * dh + dw],
                        preferred_element_type=jnp.float32)
            grp = d if grp is None else grp + d
        acc = grp if acc is None else acc + grp
    return jnp.maximum(acc + b_ref[...], 0.0)


def _repad(y, off, h0, h1, w0, w1, batch):
    rows, c = y.shape
    total = batch * _R
    z = jnp.concatenate(
        [jnp.zeros((off, c), y.dtype), y,
         jnp.zeros((total - off - rows, c), y.dtype)], axis=0)
    z4 = z.reshape(batch, _HROWS, _PITCH, c)
    hh = jax.lax.broadcasted_iota(jnp.int32, (batch, _HROWS, _PITCH, 1), 1)
    ww = jax.lax.broadcasted_iota(jnp.int32, (batch, _HROWS, _PITCH, 1), 2)
    m = (hh >= h0) & (hh < h1) & (ww >= w0) & (ww < w1)
    z4 = jnp.where(m, z4, jnp.zeros_like(z4))
    return z4.reshape(total, c)


def _convnet_kernel(x_ref, w1_ref, b1_ref, w2_ref, b2_ref, w3_ref, b3_ref,
                    w4_ref, b4_ref, o_ref, *, batch):
    total = batch * _R
    mc = total - 72
    mp2 = total - 112
    mp4 = total - 144

    x = x_ref[...].reshape(total, x_ref.shape[2])
    y1 = _conv_taps(x, w1_ref, b1_ref, mc).astype(jnp.bfloat16)

    x2 = _repad(y1, 2 * _PITCH + 2, 2, 30, 2, 30, batch)
    y2 = _conv_taps(x2, w2_ref, b2_ref, mc)
    y2p = jnp.maximum(jnp.maximum(y2[0:mp2], y2[1:mp2 + 1]),
                      jnp.maximum(y2[_PITCH:mp2 + _PITCH],
                                  y2[_PITCH + 1:mp2 + _PITCH + 1]))
    y2p = y2p.astype(jnp.bfloat16)

    x3 = _repad(y2p, _PITCH + 1, 1, 30, 1, 30, batch)
    y3 = _conv_taps(x3, w3_ref, b3_ref, mc).astype(jnp.bfloat16)

    x4 = _repad(y3, _PITCH + 1, 1, 30, 1, 30, batch)
    y4 = _conv_taps(x4, w4_ref, b4_ref, mc)
    y4p = jnp.maximum(jnp.maximum(y4[0:mp4], y4[2:mp4 + 2]),
                      jnp.maximum(y4[2 * _PITCH:mp4 + 2 * _PITCH],
                                  y4[2 * _PITCH + 2:mp4 + 2 * _PITCH + 2]))

    c4 = y4p.shape[1]
    y4z = jnp.concatenate(
        [y4p, jnp.zeros((total - mp4, c4), y4p.dtype)], axis=0)
    y6 = y4z.reshape(batch, 17, 2, 16, 2, c4)
    sub = y6[:, :14, 0, :14, 0, :]
    o_ref[...] = sub.reshape(batch, 196, c4).astype(o_ref.dtype)


def _fc_kernel(x_ref, w1_ref, b1_ref, w2_ref, b2_ref, o_ref):
    h = jnp.dot(x_ref[...], w1_ref[...], preferred_element_type=jnp.float32)
    h = jnp.maximum(h + b1_ref[...], 0.0).astype(jnp.bfloat16)
    o = jnp.dot(h, w2_ref[...], preferred_element_type=jnp.float32)
    o_ref[...] = o + b2_ref[...]


def _block_diag(w):
    t, ci, co = w.shape
    eye = jnp.eye(_L, dtype=w.dtype)
    bd = jnp.einsum("ab,tij->taibj", eye, w).reshape(t, _L * ci, _L * co)
    return bd.astype(jnp.bfloat16)


def kernel(conv1_w, conv1_b, conv2_w, conv2_b, conv3_w, conv3_b, conv4_w,
           conv4_b, fc1_w, fc1_b, fc2_w, fc2_b, x_nchw):
    n = x_nchw.shape[0]
    n4 = n // _L
    batch = 4
    assert n % (_L * batch) == 0

    xi = x_nchw.reshape(n, 28, 28).astype(jnp.float32)
    xi = jnp.pad(xi, ((0, 0), (1, 1), (1, 3)))
    xi = xi.reshape(n, 30 * 32)
    xi = jnp.pad(xi, ((0, 0), (0, _R - 30 * 32)))
    xi = xi.reshape(n4, _L, _R).transpose(0, 2, 1)
    xi = jnp.pad(xi[..., None], ((0, 0), (0, 0), (0, 0), (0, 7)))
    xi = xi.reshape(n4, _R, _L * 8).astype(jnp.bfloat16)

    w1 = _block_diag(jnp.pad(conv1_w, ((0, 0), (0, 0), (0, 7), (0, 0)))
                     .reshape(9, 8, 32))
    w2 = _block_diag(conv2_w.reshape(9, 32, 32))
    w3 = _block_diag(conv3_w.reshape(9, 32, 64))
    w4 = _block_diag(conv4_w.reshape(9, 64, 32))
    b1 = jnp.tile(conv1_b, (_L,)).reshape(1, _L * 32)
    b2 = jnp.tile(conv2_b, (_L,)).reshape(1, _L * 32)
    b3 = jnp.tile(conv3_b, (_L,)).reshape(1, _L * 64)
    b4 = jnp.tile(conv4_b, (_L,)).reshape(1, _L * 32)

    feats = pl.pallas_call(
        functools.partial(_convnet_kernel, batch=batch),
        out_shape=jax.ShapeDtypeStruct((n4, 196, _L * 32), jnp.bfloat16),
        grid_spec=pltpu.PrefetchScalarGridSpec(
            num_scalar_prefetch=0,
            grid=(n4 // batch,),
            in_specs=[
                pl.BlockSpec((batch, _R, _L * 8), lambda i: (i, 0, 0)),
                pl.BlockSpec((9, _L * 8, _L * 32), lambda i: (0, 0, 0)),
                pl.BlockSpec((1, _L * 32), lambda i: (0, 0)),
                pl.BlockSpec((9, _L * 32, _L * 32), lambda i: (0, 0, 0)),
                pl.BlockSpec((1, _L * 32), lambda i: (0, 0)),
                pl.BlockSpec((9, _L * 32, _L * 64), lambda i: (0, 0, 0)),
                pl.BlockSpec((1, _L * 64), lambda i: (0, 0)),
                pl.BlockSpec((9, _L * 64, _L * 32), lambda i: (0, 0, 0)),
                pl.BlockSpec((1, _L * 32), lambda i: (0, 0)),
            ],
            out_specs=pl.BlockSpec((batch, 196, _L * 32), lambda i: (i, 0, 0)),
        ),
        compiler_params=pltpu.CompilerParams(
            dimension_semantics=("parallel",)),
    )(xi, w1, b1, w2, b2, w3, b3, w4, b4)

    xf = feats.reshape(n4, 196, _L, 32).transpose(0, 2, 1, 3).reshape(n, 6272)
    w1p = fc1_w.reshape(32, 196, 512).transpose(1, 0, 2).reshape(6272, 512)
    w1p = w1p.astype(jnp.bfloat16)
    w2p = jnp.pad(fc2_w, ((0, 0), (0, 118))).astype(jnp.bfloat16)
    b2p = jnp.pad(fc2_b, (0, 118)).reshape(1, 128)
    b1f = fc1_b.reshape(1, 512)

    tm = min(256, n)
    assert n % tm == 0
    out = pl.pallas_call(
        _fc_kernel,
        out_shape=jax.ShapeDtypeStruct((n, 128), jnp.float32),
        grid_spec=pltpu.PrefetchScalarGridSpec(
            num_scalar_prefetch=0,
            grid=(n // tm,),
            in_specs=[
                pl.BlockSpec((tm, 6272), lambda i: (i, 0)),
                pl.BlockSpec((6272, 512), lambda i: (0, 0)),
                pl.BlockSpec((1, 512), lambda i: (0, 0)),
                pl.BlockSpec((512, 128), lambda i: (0, 0)),
                pl.BlockSpec((1, 128), lambda i: (0, 0)),
            ],
            out_specs=pl.BlockSpec((tm, 128), lambda i: (i, 0)),
        ),
        compiler_params=pltpu.CompilerParams(
            dimension_semantics=("parallel",)),
    )(xf, w1p, b1f, w2p, b2p)

    return out[:, :10]
